# Initial kernel scaffold; baseline (speedup 1.0000x reference)
import jax
import jax.numpy as jnp
from jax.experimental import pallas as pl


def kernel(edge_index, user_emb, item_emb, W_gc_0, b_gc_0, W_bi_0, b_bi_0, W_gc_1, b_gc_1, W_bi_1, b_bi_1):
    raise NotImplementedError("write your pallas kernel here")


# speedup vs baseline: 12.6224x; 12.6224x over previous
import functools

import jax
import jax.numpy as jnp
from jax import lax
from jax.experimental import pallas as pl
from jax.experimental.pallas import tpu as pltpu
from jax.experimental.pallas import tpu_sc as plsc

_HALF = 32
_CH = 128
_SUP = 4


def _make_deg_kernel(n_acc, n_rows):
  rt = n_rows // 16
  nsup = rt // _SUP
  zrows = n_acc // 16
  mesh = plsc.VectorSubcoreMesh(core_axis_name="c", subcore_axis_name="s")

  @functools.partial(
      pl.kernel,
      out_type=jax.ShapeDtypeStruct((n_acc, 8), jnp.float32),
      mesh=mesh,
      compiler_params=pltpu.CompilerParams(use_tc_tiling_on_sc=False),
      scratch_types=[
          pltpu.VMEM((_SUP, _CH), jnp.int32),
          pltpu.VMEM((_CH, 8), jnp.float32),
          pltpu.VMEM_SHARED((n_acc, 8), jnp.float32),
          pltpu.SemaphoreType.DMA,
      ],
  )
  def deg_kernel(rows2d, zeros8_hbm, ones_hbm, out, rowbuf, onesbuf, degsh, sem):
    del sem
    c = lax.axis_index("c")
    s = lax.axis_index("s")
    pltpu.sync_copy(
        zeros8_hbm.at[pl.ds(s * zrows, zrows)],
        degsh.at[pl.ds(s * zrows, zrows)],
    )
    pltpu.sync_copy(ones_hbm, onesbuf)
    plsc.subcore_barrier()

    def body(i, carry):
      r = s * rt + i * _SUP
      pltpu.sync_copy(rows2d.at[pl.ds(r, _SUP)], rowbuf)
      for j in range(_SUP):
        pltpu.sync_copy(onesbuf, degsh.at[rowbuf.at[j]], add=True)
      return carry

    lax.fori_loop(0, nsup, body, 0)
    plsc.subcore_barrier()

    @pl.when(c == 0)
    def _():
      pltpu.sync_copy(
          degsh.at[pl.ds(s * zrows, zrows)], out.at[pl.ds(s * zrows, zrows)]
      )

  return deg_kernel


def _make_agg_kernel(n_acc, n_rows):
  rt = n_rows // 16
  nsup = rt // _SUP
  zrows = n_acc // 16
  mesh = plsc.VectorSubcoreMesh(core_axis_name="c", subcore_axis_name="s")

  @functools.partial(
      pl.kernel,
      out_type=jax.ShapeDtypeStruct((2, n_acc, _HALF), jnp.float32),
      mesh=mesh,
      compiler_params=pltpu.CompilerParams(use_tc_tiling_on_sc=False),
      scratch_types=[
          pltpu.VMEM((_SUP, _CH), jnp.int32),
          pltpu.VMEM((_SUP, _CH), jnp.int32),
          pltpu.VMEM((_SUP, _CH, _HALF), jnp.float32),
          pltpu.VMEM_SHARED((n_acc, _HALF), jnp.float32),
          pltpu.SemaphoreType.DMA,
      ],
  )
  def agg_kernel(
      ego_flat, cols3d, rows2d, zeros_hbm, out, colbuf, rowbuf, gbuf, aggsh, sem
  ):
    c = lax.axis_index("c")
    s = lax.axis_index("s")
    pltpu.sync_copy(
        zeros_hbm.at[pl.ds(s * zrows, zrows)], aggsh.at[pl.ds(s * zrows, zrows)]
    )
    plsc.subcore_barrier()

    def body(i, carry):
      r = s * rt + i * _SUP
      pltpu.sync_copy(cols3d.at[c].at[pl.ds(r, _SUP)], colbuf)
      pltpu.sync_copy(rows2d.at[pl.ds(r, _SUP)], rowbuf)
      for j in range(_SUP):
        pltpu.async_copy(ego_flat.at[colbuf.at[j]], gbuf.at[j], sem).wait()
        pltpu.sync_copy(gbuf.at[j], aggsh.at[rowbuf.at[j]], add=True)
      return carry

    lax.fori_loop(0, nsup, body, 0)
    plsc.subcore_barrier()
    pltpu.sync_copy(
        aggsh.at[pl.ds(s * zrows, zrows)], out.at[c].at[pl.ds(s * zrows, zrows)]
    )

  return agg_kernel


def _tc_update(ego_split, agg_split, deg8, wg, bg, wb, bb, block):
  n = ego_split.shape[1]

  def body(ego_ref, agg_ref, deg_ref, wg_ref, bg_ref, wb_ref, bb_ref, out_ref):
    ego = jnp.concatenate([ego_ref[0], ego_ref[1]], axis=1)
    agg = jnp.concatenate([agg_ref[0], agg_ref[1]], axis=1)
    inv = 1.0 / (deg_ref[:, 0:1] + 1.0)
    side = (agg + ego) * inv
    se = jnp.dot(side, wg_ref[...], preferred_element_type=jnp.float32) + bg_ref[...]
    se = jnp.where(se >= 0.0, se, 0.01 * se)
    be = (
        jnp.dot(ego * side, wb_ref[...], preferred_element_type=jnp.float32)
        + bb_ref[...]
    )
    be = jnp.where(be >= 0.0, be, 0.01 * be)
    e = se + be
    nrm = jnp.sqrt(jnp.sum(e * e, axis=1, keepdims=True))
    nrm = jnp.maximum(nrm, 1e-12)
    o = e / nrm
    out_ref[0] = o[:, :_HALF]
    out_ref[1] = o[:, _HALF:]

  return pl.pallas_call(
      body,
      grid=(n // block,),
      in_specs=[
          pl.BlockSpec((2, block, _HALF), lambda i: (0, i, 0)),
          pl.BlockSpec((2, block, _HALF), lambda i: (0, i, 0)),
          pl.BlockSpec((block, 8), lambda i: (i, 0)),
          pl.BlockSpec((64, 64), lambda i: (0, 0)),
          pl.BlockSpec((1, 64), lambda i: (0, 0)),
          pl.BlockSpec((64, 64), lambda i: (0, 0)),
          pl.BlockSpec((1, 64), lambda i: (0, 0)),
      ],
      out_specs=pl.BlockSpec((2, block, _HALF), lambda i: (0, i, 0)),
      out_shape=jax.ShapeDtypeStruct((2, n, _HALF), jnp.float32),
  )(ego_split, agg_split, deg8, wg, bg, wb, bb)


def kernel(edge_index, user_emb, item_emb, W_gc_0, b_gc_0, W_bi_0, b_bi_0,
           W_gc_1, b_gc_1, W_bi_1, b_bi_1):
  n = user_emb.shape[0] + item_emb.shape[0]
  e = edge_index.shape[1]

  ep = -(-e // 16384) * 16384
  pad = ep - e
  rows_p = jnp.concatenate(
      [edge_index[0], jnp.full((pad,), n, jnp.int32)])
  cols_p = jnp.concatenate([edge_index[1], jnp.zeros((pad,), jnp.int32)])
  n_rows = ep // _CH
  rows2d = rows_p.reshape(n_rows, _CH)
  cols2d = cols_p.reshape(n_rows, _CH)
  cols3d = jnp.stack([cols2d, cols2d + n])

  n_acc = -(-(n + 1) // 128) * 128
  zeros_hbm = jnp.zeros((n_acc, _HALF), jnp.float32)
  zeros8_hbm = jnp.zeros((n_acc, 8), jnp.float32)
  ones_hbm = jnp.ones((_CH, 8), jnp.float32)

  ego0 = jnp.concatenate([user_emb, item_emb], axis=0)
  ego0_split = jnp.stack([ego0[:, :_HALF], ego0[:, _HALF:]])

  deg_k = _make_deg_kernel(n_acc, n_rows)
  agg_k = _make_agg_kernel(n_acc, n_rows)

  deg8 = deg_k(rows2d, zeros8_hbm, ones_hbm)

  agg0 = agg_k(ego0_split.reshape(2 * n, _HALF), cols3d, rows2d, zeros_hbm)
  ego1_split = _tc_update(ego0_split, agg0, deg8, W_gc_0, b_gc_0, W_bi_0,
                          b_bi_0, block=2000)
  agg1 = agg_k(ego1_split.reshape(2 * n, _HALF), cols3d, rows2d, zeros_hbm)
  ego2_split = _tc_update(ego1_split, agg1, deg8, W_gc_1, b_gc_1, W_bi_1,
                          b_bi_1, block=2000)

  def unsplit(x):
    return jnp.concatenate([x[0], x[1]], axis=1)

  return jnp.concatenate([ego0, unsplit(ego1_split), unsplit(ego2_split)],
                         axis=1)


# speedup vs baseline: 16.6912x; 1.3223x over previous
import functools

import jax
import jax.numpy as jnp
from jax import lax
from jax.experimental import pallas as pl
from jax.experimental.pallas import tpu as pltpu
from jax.experimental.pallas import tpu_sc as plsc

_HALF = 32
_CH = 128
_SUP = 4


def _make_deg_kernel(n_acc, n_rows):
  rt = n_rows // 16
  nsup = rt // _SUP
  zrows = n_acc // 16
  mesh = plsc.VectorSubcoreMesh(core_axis_name="c", subcore_axis_name="s")

  @functools.partial(
      pl.kernel,
      out_type=jax.ShapeDtypeStruct((n_acc, 8), jnp.float32),
      mesh=mesh,
      compiler_params=pltpu.CompilerParams(use_tc_tiling_on_sc=False),
      scratch_types=[
          pltpu.VMEM((_SUP * _CH,), jnp.int32),
          pltpu.VMEM((_SUP * _CH, 8), jnp.float32),
          pltpu.VMEM_SHARED((n_acc, 8), jnp.float32),
          pltpu.SemaphoreType.DMA,
      ],
  )
  def deg_kernel(rows_flat, zeros8_hbm, ones_hbm, out, rowbuf, onesbuf, degsh, sem):
    del sem
    c = lax.axis_index("c")
    s = lax.axis_index("s")
    pltpu.sync_copy(
        zeros8_hbm.at[pl.ds(s * zrows, zrows)],
        degsh.at[pl.ds(s * zrows, zrows)],
    )
    pltpu.sync_copy(ones_hbm, onesbuf)
    plsc.subcore_barrier()

    def body(i, carry):
      r = (s * rt + i * _SUP) * _CH
      pltpu.sync_copy(rows_flat.at[pl.ds(r, _SUP * _CH)], rowbuf)
      pltpu.sync_copy(onesbuf, degsh.at[rowbuf], add=True)
      return carry

    lax.fori_loop(0, nsup, body, 0)
    plsc.subcore_barrier()

    @pl.when(c == 0)
    def _():
      pltpu.sync_copy(
          degsh.at[pl.ds(s * zrows, zrows)], out.at[pl.ds(s * zrows, zrows)]
      )

  return deg_kernel


def _make_agg_kernel(n_acc, n_rows):
  rt = n_rows // 16
  nsup = rt // _SUP
  zrows = n_acc // 16
  mesh = plsc.VectorSubcoreMesh(core_axis_name="c", subcore_axis_name="s")

  @functools.partial(
      pl.kernel,
      out_type=jax.ShapeDtypeStruct((2, n_acc, _HALF), jnp.float32),
      mesh=mesh,
      compiler_params=pltpu.CompilerParams(use_tc_tiling_on_sc=False),
      scratch_types=[
          pltpu.VMEM((_SUP * _CH,), jnp.int32),
          pltpu.VMEM((_SUP * _CH,), jnp.int32),
          pltpu.VMEM((_SUP * _CH, _HALF), jnp.float32),
          pltpu.VMEM_SHARED((n_acc, _HALF), jnp.float32),
          pltpu.SemaphoreType.DMA,
      ],
  )
  def agg_kernel(
      ego_flat, cols2f, rows_flat, zeros_hbm, out, colbuf, rowbuf, gbuf, aggsh, sem
  ):
    c = lax.axis_index("c")
    s = lax.axis_index("s")
    pltpu.sync_copy(
        zeros_hbm.at[pl.ds(s * zrows, zrows)], aggsh.at[pl.ds(s * zrows, zrows)]
    )
    plsc.subcore_barrier()

    def body(i, carry):
      r = (s * rt + i * _SUP) * _CH
      pltpu.sync_copy(cols2f.at[c].at[pl.ds(r, _SUP * _CH)], colbuf)
      pltpu.sync_copy(rows_flat.at[pl.ds(r, _SUP * _CH)], rowbuf)
      pltpu.async_copy(ego_flat.at[colbuf], gbuf, sem).wait()
      pltpu.sync_copy(gbuf, aggsh.at[rowbuf], add=True)
      return carry

    lax.fori_loop(0, nsup, body, 0)
    plsc.subcore_barrier()
    pltpu.sync_copy(
        aggsh.at[pl.ds(s * zrows, zrows)], out.at[c].at[pl.ds(s * zrows, zrows)]
    )

  return agg_kernel


def _tc_update(ego_split, agg_split, deg8, wg, bg, wb, bb, block):
  n = ego_split.shape[1]

  def body(ego_ref, agg_ref, deg_ref, wg_ref, bg_ref, wb_ref, bb_ref, out_ref):
    ego = jnp.concatenate([ego_ref[0], ego_ref[1]], axis=1)
    agg = jnp.concatenate([agg_ref[0], agg_ref[1]], axis=1)
    inv = 1.0 / (deg_ref[:, 0:1] + 1.0)
    side = (agg + ego) * inv
    se = jnp.dot(side, wg_ref[...], preferred_element_type=jnp.float32) + bg_ref[...]
    se = jnp.where(se >= 0.0, se, 0.01 * se)
    be = (
        jnp.dot(ego * side, wb_ref[...], preferred_element_type=jnp.float32)
        + bb_ref[...]
    )
    be = jnp.where(be >= 0.0, be, 0.01 * be)
    e = se + be
    nrm = jnp.sqrt(jnp.sum(e * e, axis=1, keepdims=True))
    nrm = jnp.maximum(nrm, 1e-12)
    o = e / nrm
    out_ref[0] = o[:, :_HALF]
    out_ref[1] = o[:, _HALF:]

  return pl.pallas_call(
      body,
      grid=(n // block,),
      in_specs=[
          pl.BlockSpec((2, block, _HALF), lambda i: (0, i, 0)),
          pl.BlockSpec((2, block, _HALF), lambda i: (0, i, 0)),
          pl.BlockSpec((block, 8), lambda i: (i, 0)),
          pl.BlockSpec((64, 64), lambda i: (0, 0)),
          pl.BlockSpec((1, 64), lambda i: (0, 0)),
          pl.BlockSpec((64, 64), lambda i: (0, 0)),
          pl.BlockSpec((1, 64), lambda i: (0, 0)),
      ],
      out_specs=pl.BlockSpec((2, block, _HALF), lambda i: (0, i, 0)),
      out_shape=jax.ShapeDtypeStruct((2, n, _HALF), jnp.float32),
  )(ego_split, agg_split, deg8, wg, bg, wb, bb)


def kernel(edge_index, user_emb, item_emb, W_gc_0, b_gc_0, W_bi_0, b_bi_0,
           W_gc_1, b_gc_1, W_bi_1, b_bi_1):
  n = user_emb.shape[0] + item_emb.shape[0]
  e = edge_index.shape[1]

  ep = -(-e // 16384) * 16384
  pad = ep - e
  rows_p = jnp.concatenate(
      [edge_index[0], jnp.full((pad,), n, jnp.int32)])
  cols_p = jnp.concatenate([edge_index[1], jnp.zeros((pad,), jnp.int32)])
  n_rows = ep // _CH
  rows_flat = rows_p
  cols2f = jnp.stack([cols_p, cols_p + n])

  n_acc = -(-(n + 1) // 128) * 128
  zeros_hbm = jnp.zeros((n_acc, _HALF), jnp.float32)
  zeros8_hbm = jnp.zeros((n_acc, 8), jnp.float32)
  ones_hbm = jnp.ones((_SUP * _CH, 8), jnp.float32)

  ego0 = jnp.concatenate([user_emb, item_emb], axis=0)
  ego0_split = jnp.stack([ego0[:, :_HALF], ego0[:, _HALF:]])

  deg_k = _make_deg_kernel(n_acc, n_rows)
  agg_k = _make_agg_kernel(n_acc, n_rows)

  deg8 = deg_k(rows_flat, zeros8_hbm, ones_hbm)

  agg0 = agg_k(ego0_split.reshape(2 * n, _HALF), cols2f, rows_flat, zeros_hbm)
  ego1_split = _tc_update(ego0_split, agg0, deg8, W_gc_0, b_gc_0, W_bi_0,
                          b_bi_0, block=2000)
  agg1 = agg_k(ego1_split.reshape(2 * n, _HALF), cols2f, rows_flat, zeros_hbm)
  ego2_split = _tc_update(ego1_split, agg1, deg8, W_gc_1, b_gc_1, W_bi_1,
                          b_bi_1, block=2000)

  def unsplit(x):
    return jnp.concatenate([x[0], x[1]], axis=1)

  return jnp.concatenate([ego0, unsplit(ego1_split), unsplit(ego2_split)],
                         axis=1)


# speedup vs baseline: 21.6181x; 1.2952x over previous
import functools

import jax
import jax.numpy as jnp
from jax import lax
from jax.experimental import pallas as pl
from jax.experimental.pallas import tpu as pltpu
from jax.experimental.pallas import tpu_sc as plsc

_HALF = 32
_DCH = 512


def _pick_chunk(ept):
  best = 8
  for d in range(8, 417, 8):
    if ept % d == 0:
      best = d
  return best


def _make_deg_kernel(n_acc, ep):
  ept = ep // 32
  nit = ept // _DCH
  zrows = n_acc // 16
  mesh = plsc.VectorSubcoreMesh(core_axis_name="c", subcore_axis_name="s")

  @functools.partial(
      pl.kernel,
      out_type=jax.ShapeDtypeStruct((2, n_acc, 8), jnp.float32),
      mesh=mesh,
      compiler_params=pltpu.CompilerParams(use_tc_tiling_on_sc=False),
      scratch_types=[
          pltpu.VMEM((_DCH,), jnp.int32),
          pltpu.VMEM((_DCH, 8), jnp.float32),
          pltpu.VMEM_SHARED((n_acc, 8), jnp.float32),
          pltpu.SemaphoreType.DMA,
      ],
  )
  def deg_kernel(rows_flat, zeros8_hbm, ones_hbm, out, rowbuf, onesbuf, degsh, sem):
    del sem
    c = lax.axis_index("c")
    s = lax.axis_index("s")
    pltpu.sync_copy(
        zeros8_hbm.at[pl.ds(s * zrows, zrows)],
        degsh.at[pl.ds(s * zrows, zrows)],
    )
    pltpu.sync_copy(ones_hbm, onesbuf)
    plsc.subcore_barrier()

    def body(i, carry):
      r = (c * 16 + s) * ept + i * _DCH
      pltpu.sync_copy(rows_flat.at[pl.ds(r, _DCH)], rowbuf)
      pltpu.sync_copy(onesbuf, degsh.at[rowbuf], add=True)
      return carry

    lax.fori_loop(0, nit, body, 0)
    plsc.subcore_barrier()
    pltpu.sync_copy(
        degsh.at[pl.ds(s * zrows, zrows)], out.at[c].at[pl.ds(s * zrows, zrows)]
    )

  return deg_kernel


def _make_agg_kernel(n_acc, ep):
  ept = ep // 16
  chunk = _pick_chunk(ept)
  nit = ept // chunk
  zrows = n_acc // 16
  cbytes = chunk * _HALF * 4
  ibytes = chunk * 4
  mesh = plsc.VectorSubcoreMesh(core_axis_name="c", subcore_axis_name="s")

  @functools.partial(
      pl.kernel,
      out_type=jax.ShapeDtypeStruct((2, n_acc, _HALF), jnp.float32),
      mesh=mesh,
      compiler_params=pltpu.CompilerParams(use_tc_tiling_on_sc=False),
      scratch_types=[
          pltpu.VMEM((3, chunk), jnp.int32),
          pltpu.VMEM((3, chunk), jnp.int32),
          pltpu.VMEM((2, chunk, _HALF), jnp.float32),
          pltpu.VMEM_SHARED((n_acc, _HALF), jnp.float32),
          pltpu.SemaphoreType.DMA,
          pltpu.SemaphoreType.DMA,
          pltpu.SemaphoreType.DMA,
      ],
  )
  def agg_kernel(ego_flat, cols2f, rows_flat, zeros_hbm, out,
                 colbuf, rowbuf, gbuf, aggsh, semi, semg, sems):
    c = lax.axis_index("c")
    s = lax.axis_index("s")
    base = s * ept
    pltpu.sync_copy(
        zeros_hbm.at[pl.ds(s * zrows, zrows)], aggsh.at[pl.ds(s * zrows, zrows)]
    )
    plsc.subcore_barrier()

    def fire_idx(g, slot):
      pltpu.async_copy(
          cols2f.at[c].at[pl.ds(base + g * chunk, chunk)], colbuf.at[slot], semi
      )
      pltpu.async_copy(
          rows_flat.at[pl.ds(base + g * chunk, chunk)], rowbuf.at[slot], semi
      )

    def drain_idx(dst, sem):
      pltpu.make_async_copy(rows_flat.at[pl.ds(0, chunk)], dst, sem).wait()

    fire_idx(0, 0)

    def body(g, carry):
      b2 = lax.rem(g, 2)
      s3 = lax.rem(g, 3)

      @pl.when(g >= 2)
      def _():
        pltpu.make_async_copy(
            zeros_hbm.at[pl.ds(0, chunk)], gbuf.at[0], sems
        ).wait()

      drain_idx(colbuf.at[s3], semi)
      drain_idx(rowbuf.at[s3], semi)

      @pl.when(g + 1 < nit)
      def _():
        fire_idx(g + 1, lax.rem(g + 1, 3))

      pltpu.async_copy(ego_flat.at[colbuf.at[s3]], gbuf.at[b2], semg).wait()

      pltpu.async_copy(gbuf.at[b2], aggsh.at[rowbuf.at[s3]], sems, add=True)
      return carry

    lax.fori_loop(0, nit, body, 0)
    pltpu.make_async_copy(zeros_hbm.at[pl.ds(0, chunk)], gbuf.at[0], sems).wait()
    pltpu.make_async_copy(zeros_hbm.at[pl.ds(0, chunk)], gbuf.at[1], sems).wait()

    plsc.subcore_barrier()
    pltpu.sync_copy(
        aggsh.at[pl.ds(s * zrows, zrows)], out.at[c].at[pl.ds(s * zrows, zrows)]
    )

  return agg_kernel


def _tc_update(ego_split, agg_split, deg8, wg, bg, wb, bb, block):
  n = ego_split.shape[1]

  def body(ego_ref, agg_ref, deg_ref, wg_ref, bg_ref, wb_ref, bb_ref, out_ref):
    ego = jnp.concatenate([ego_ref[0], ego_ref[1]], axis=1)
    agg = jnp.concatenate([agg_ref[0], agg_ref[1]], axis=1)
    deg = deg_ref[0, :, 0:1] + deg_ref[1, :, 0:1]
    inv = 1.0 / (deg + 1.0)
    side = (agg + ego) * inv
    se = jnp.dot(side, wg_ref[...], preferred_element_type=jnp.float32) + bg_ref[...]
    se = jnp.where(se >= 0.0, se, 0.01 * se)
    be = (
        jnp.dot(ego * side, wb_ref[...], preferred_element_type=jnp.float32)
        + bb_ref[...]
    )
    be = jnp.where(be >= 0.0, be, 0.01 * be)
    e = se + be
    nrm = jnp.sqrt(jnp.sum(e * e, axis=1, keepdims=True))
    nrm = jnp.maximum(nrm, 1e-12)
    o = e / nrm
    out_ref[0] = o[:, :_HALF]
    out_ref[1] = o[:, _HALF:]

  return pl.pallas_call(
      body,
      grid=(n // block,),
      in_specs=[
          pl.BlockSpec((2, block, _HALF), lambda i: (0, i, 0)),
          pl.BlockSpec((2, block, _HALF), lambda i: (0, i, 0)),
          pl.BlockSpec((2, block, 8), lambda i: (0, i, 0)),
          pl.BlockSpec((64, 64), lambda i: (0, 0)),
          pl.BlockSpec((1, 64), lambda i: (0, 0)),
          pl.BlockSpec((64, 64), lambda i: (0, 0)),
          pl.BlockSpec((1, 64), lambda i: (0, 0)),
      ],
      out_specs=pl.BlockSpec((2, block, _HALF), lambda i: (0, i, 0)),
      out_shape=jax.ShapeDtypeStruct((2, n, _HALF), jnp.float32),
  )(ego_split, agg_split, deg8, wg, bg, wb, bb)


def kernel(edge_index, user_emb, item_emb, W_gc_0, b_gc_0, W_bi_0, b_bi_0,
           W_gc_1, b_gc_1, W_bi_1, b_bi_1):
  n = user_emb.shape[0] + item_emb.shape[0]
  e = edge_index.shape[1]

  ep = -(-e // 16384) * 16384
  pad = ep - e
  rows_flat = jnp.concatenate(
      [edge_index[0], jnp.full((pad,), n, jnp.int32)])
  cols_p = jnp.concatenate([edge_index[1], jnp.zeros((pad,), jnp.int32)])
  cols2f = jnp.stack([cols_p, cols_p + n])

  n_acc = -(-(n + 1) // 128) * 128
  zeros_hbm = jnp.zeros((n_acc, _HALF), jnp.float32)
  zeros8_hbm = jnp.zeros((n_acc, 8), jnp.float32)
  ones_hbm = jnp.ones((_DCH, 8), jnp.float32)

  ego0 = jnp.concatenate([user_emb, item_emb], axis=0)
  ego0_split = jnp.stack([ego0[:, :_HALF], ego0[:, _HALF:]])

  deg_k = _make_deg_kernel(n_acc, ep)
  agg_k = _make_agg_kernel(n_acc, ep)

  deg8 = deg_k(rows_flat, zeros8_hbm, ones_hbm)

  agg0 = agg_k(ego0_split.reshape(2 * n, _HALF), cols2f, rows_flat, zeros_hbm)
  ego1_split = _tc_update(ego0_split, agg0, deg8, W_gc_0, b_gc_0, W_bi_0,
                          b_bi_0, block=2000)
  agg1 = agg_k(ego1_split.reshape(2 * n, _HALF), cols2f, rows_flat, zeros_hbm)
  ego2_split = _tc_update(ego1_split, agg1, deg8, W_gc_1, b_gc_1, W_bi_1,
                          b_bi_1, block=2000)

  def unsplit(x):
    return jnp.concatenate([x[0], x[1]], axis=1)

  return jnp.concatenate([ego0, unsplit(ego1_split), unsplit(ego2_split)],
                         axis=1)


# speedup vs baseline: 24.6276x; 1.1392x over previous
import functools

import jax
import jax.numpy as jnp
from jax import lax
from jax.experimental import pallas as pl
from jax.experimental.pallas import tpu as pltpu
from jax.experimental.pallas import tpu_sc as plsc

_HALF = 32
_DCH = 512


def _pick_chunk(ept):
  best = 8
  for d in range(8, 417, 8):
    if ept % d == 0:
      best = d
  return best


def _make_deg_kernel(n_acc, ep):
  ept = ep // 32
  nit = ept // _DCH
  zrows = n_acc // 16
  mesh = plsc.VectorSubcoreMesh(core_axis_name="c", subcore_axis_name="s")

  @functools.partial(
      pl.kernel,
      out_type=jax.ShapeDtypeStruct((2, n_acc, 8), jnp.float32),
      mesh=mesh,
      compiler_params=pltpu.CompilerParams(use_tc_tiling_on_sc=False),
      scratch_types=[
          pltpu.VMEM((_DCH,), jnp.int32),
          pltpu.VMEM((_DCH, 8), jnp.float32),
          pltpu.VMEM_SHARED((n_acc, 8), jnp.float32),
          pltpu.SemaphoreType.DMA,
      ],
  )
  def deg_kernel(rows_flat, zeros8_hbm, ones_hbm, out, rowbuf, onesbuf, degsh, sem):
    del sem
    c = lax.axis_index("c")
    s = lax.axis_index("s")
    pltpu.sync_copy(
        zeros8_hbm.at[pl.ds(s * zrows, zrows)],
        degsh.at[pl.ds(s * zrows, zrows)],
    )
    pltpu.sync_copy(ones_hbm, onesbuf)
    plsc.subcore_barrier()

    def body(i, carry):
      r = (c * 16 + s) * ept + i * _DCH
      pltpu.sync_copy(rows_flat.at[pl.ds(r, _DCH)], rowbuf)
      pltpu.sync_copy(onesbuf, degsh.at[rowbuf], add=True)
      return carry

    lax.fori_loop(0, nit, body, 0)
    plsc.subcore_barrier()
    pltpu.sync_copy(
        degsh.at[pl.ds(s * zrows, zrows)], out.at[c].at[pl.ds(s * zrows, zrows)]
    )

  return deg_kernel


def _make_agg_kernel(n_acc, ep):
  ept = ep // 16
  chunk = _pick_chunk(ept)
  nit = ept // chunk
  zrows = n_acc // 16
  cbytes = chunk * _HALF * 4
  ibytes = chunk * 4
  mesh = plsc.VectorSubcoreMesh(core_axis_name="c", subcore_axis_name="s")

  @functools.partial(
      pl.kernel,
      out_type=jax.ShapeDtypeStruct((n_acc, 2 * _HALF), jnp.float32),
      mesh=mesh,
      compiler_params=pltpu.CompilerParams(use_tc_tiling_on_sc=False),
      scratch_types=[
          pltpu.VMEM((3, chunk), jnp.int32),
          pltpu.VMEM((3, chunk), jnp.int32),
          pltpu.VMEM((2, chunk, _HALF), jnp.float32),
          pltpu.VMEM_SHARED((n_acc, _HALF), jnp.float32),
          pltpu.SemaphoreType.DMA,
          pltpu.SemaphoreType.DMA,
          pltpu.SemaphoreType.DMA,
      ],
  )
  def agg_kernel(ego_flat, cols2f, rows_flat, zeros_hbm, out,
                 colbuf, rowbuf, gbuf, aggsh, semi, semg, sems):
    c = lax.axis_index("c")
    s = lax.axis_index("s")
    base = s * ept
    pltpu.sync_copy(
        zeros_hbm.at[pl.ds(s * zrows, zrows)], aggsh.at[pl.ds(s * zrows, zrows)]
    )
    plsc.subcore_barrier()

    def fire_idx(g, slot):
      pltpu.async_copy(
          cols2f.at[c].at[pl.ds(base + g * chunk, chunk)], colbuf.at[slot], semi
      )
      pltpu.async_copy(
          rows_flat.at[pl.ds(base + g * chunk, chunk)], rowbuf.at[slot], semi
      )

    def drain_idx(dst, sem):
      pltpu.make_async_copy(rows_flat.at[pl.ds(0, chunk)], dst, sem).wait()

    fire_idx(0, 0)

    def body(g, carry):
      b2 = lax.rem(g, 2)
      s3 = lax.rem(g, 3)

      @pl.when(g >= 2)
      def _():
        pltpu.make_async_copy(
            zeros_hbm.at[pl.ds(0, chunk)], gbuf.at[0], sems
        ).wait()

      drain_idx(colbuf.at[s3], semi)
      drain_idx(rowbuf.at[s3], semi)

      @pl.when(g + 1 < nit)
      def _():
        fire_idx(g + 1, lax.rem(g + 1, 3))

      pltpu.async_copy(ego_flat.at[colbuf.at[s3]], gbuf.at[b2], semg).wait()

      pltpu.async_copy(gbuf.at[b2], aggsh.at[rowbuf.at[s3]], sems, add=True)
      return carry

    lax.fori_loop(0, nit, body, 0)
    pltpu.make_async_copy(zeros_hbm.at[pl.ds(0, chunk)], gbuf.at[0], sems).wait()
    pltpu.make_async_copy(zeros_hbm.at[pl.ds(0, chunk)], gbuf.at[1], sems).wait()

    plsc.subcore_barrier()
    pltpu.sync_copy(
        aggsh.at[pl.ds(s * zrows, zrows)],
        out.at[pl.ds(s * zrows, zrows), pl.ds(c * _HALF, _HALF)],
    )

  return agg_kernel


def _layer_math(ego, agg, deg, wg_ref, bg_ref, wb_ref, bb_ref):
  inv = 1.0 / (deg + 1.0)
  side = (agg + ego) * inv
  se = jnp.dot(side, wg_ref[...], preferred_element_type=jnp.float32) + bg_ref[...]
  se = jnp.where(se >= 0.0, se, 0.01 * se)
  be = (
      jnp.dot(ego * side, wb_ref[...], preferred_element_type=jnp.float32)
      + bb_ref[...]
  )
  be = jnp.where(be >= 0.0, be, 0.01 * be)
  e = se + be
  nrm = jnp.sqrt(jnp.sum(e * e, axis=1, keepdims=True))
  nrm = jnp.maximum(nrm, 1e-12)
  return e / nrm


def _tc_update(ego, agg, deg8, wg, bg, wb, bb, block):
  n = ego.shape[0]

  def body(ego_ref, agg_ref, deg_ref, wg_ref, bg_ref, wb_ref, bb_ref, out_ref):
    deg = deg_ref[0, :, 0:1] + deg_ref[1, :, 0:1]
    out_ref[...] = _layer_math(
        ego_ref[...], agg_ref[...], deg, wg_ref, bg_ref, wb_ref, bb_ref)

  return pl.pallas_call(
      body,
      grid=(n // block,),
      in_specs=[
          pl.BlockSpec((block, 64), lambda i: (i, 0)),
          pl.BlockSpec((block, 64), lambda i: (i, 0)),
          pl.BlockSpec((2, block, 8), lambda i: (0, i, 0)),
          pl.BlockSpec((64, 64), lambda i: (0, 0)),
          pl.BlockSpec((1, 64), lambda i: (0, 0)),
          pl.BlockSpec((64, 64), lambda i: (0, 0)),
          pl.BlockSpec((1, 64), lambda i: (0, 0)),
      ],
      out_specs=pl.BlockSpec((block, 64), lambda i: (i, 0)),
      out_shape=jax.ShapeDtypeStruct((n, 64), jnp.float32),
  )(ego, agg, deg8, wg, bg, wb, bb)


def _tc_update_final(ego0, ego1, agg, deg8, wg, bg, wb, bb, block):
  n = ego1.shape[0]

  def body(e0_ref, e1_ref, agg_ref, deg_ref, wg_ref, bg_ref, wb_ref, bb_ref,
           out_ref):
    deg = deg_ref[0, :, 0:1] + deg_ref[1, :, 0:1]
    o = _layer_math(
        e1_ref[...], agg_ref[...], deg, wg_ref, bg_ref, wb_ref, bb_ref)
    out_ref[...] = jnp.concatenate([e0_ref[...], e1_ref[...], o], axis=1)

  return pl.pallas_call(
      body,
      grid=(n // block,),
      in_specs=[
          pl.BlockSpec((block, 64), lambda i: (i, 0)),
          pl.BlockSpec((block, 64), lambda i: (i, 0)),
          pl.BlockSpec((block, 64), lambda i: (i, 0)),
          pl.BlockSpec((2, block, 8), lambda i: (0, i, 0)),
          pl.BlockSpec((64, 64), lambda i: (0, 0)),
          pl.BlockSpec((1, 64), lambda i: (0, 0)),
          pl.BlockSpec((64, 64), lambda i: (0, 0)),
          pl.BlockSpec((1, 64), lambda i: (0, 0)),
      ],
      out_specs=pl.BlockSpec((block, 192), lambda i: (i, 0)),
      out_shape=jax.ShapeDtypeStruct((n, 192), jnp.float32),
  )(ego0, ego1, agg, deg8, wg, bg, wb, bb)


def kernel(edge_index, user_emb, item_emb, W_gc_0, b_gc_0, W_bi_0, b_bi_0,
           W_gc_1, b_gc_1, W_bi_1, b_bi_1):
  n = user_emb.shape[0] + item_emb.shape[0]
  e = edge_index.shape[1]

  ep = -(-e // 16384) * 16384
  pad = ep - e
  rows_flat = jnp.concatenate(
      [edge_index[0], jnp.full((pad,), n, jnp.int32)])
  cols_p = jnp.concatenate([edge_index[1], jnp.zeros((pad,), jnp.int32)])
  cols2f = jnp.stack([2 * cols_p, 2 * cols_p + 1])

  n_acc = -(-(n + 1) // 128) * 128
  zeros_hbm = jnp.zeros((n_acc, _HALF), jnp.float32)
  zeros8_hbm = jnp.zeros((n_acc, 8), jnp.float32)
  ones_hbm = jnp.ones((_DCH, 8), jnp.float32)

  ego0 = jnp.concatenate([user_emb, item_emb], axis=0)

  deg_k = _make_deg_kernel(n_acc, ep)
  agg_k = _make_agg_kernel(n_acc, ep)

  deg8 = deg_k(rows_flat, zeros8_hbm, ones_hbm)

  agg0 = agg_k(ego0.reshape(2 * n, _HALF), cols2f, rows_flat, zeros_hbm)
  ego1 = _tc_update(ego0, agg0, deg8, W_gc_0, b_gc_0, W_bi_0, b_bi_0,
                    block=2000)
  agg1 = agg_k(ego1.reshape(2 * n, _HALF), cols2f, rows_flat, zeros_hbm)
  return _tc_update_final(ego0, ego1, agg1, deg8, W_gc_1, b_gc_1, W_bi_1,
                          b_bi_1, block=2000)


# speedup vs baseline: 28.6513x; 1.1634x over previous
import functools

import jax
import jax.numpy as jnp
from jax import lax
from jax.experimental import pallas as pl
from jax.experimental.pallas import tpu as pltpu
from jax.experimental.pallas import tpu_sc as plsc

_HALF = 32


def _pick_chunk(ept, cap):
  best = 8
  for d in range(8, cap + 1, 8):
    if ept % d == 0:
      best = d
  return best


def _make_deg_kernel(n_acc, ep):
  ept = ep // 32
  dch = _pick_chunk(ept, 512)
  nit = ept // dch
  zrows = n_acc // 16
  mesh = plsc.VectorSubcoreMesh(core_axis_name="c", subcore_axis_name="s")

  @functools.partial(
      pl.kernel,
      out_type=jax.ShapeDtypeStruct((2, n_acc, 8), jnp.float32),
      mesh=mesh,
      compiler_params=pltpu.CompilerParams(use_tc_tiling_on_sc=False),
      scratch_types=[
          pltpu.VMEM((dch,), jnp.int32),
          pltpu.VMEM((dch, 8), jnp.float32),
          pltpu.VMEM_SHARED((n_acc, 8), jnp.float32),
          pltpu.SemaphoreType.DMA,
      ],
  )
  def deg_kernel(rows_flat, zeros8_hbm, ones_hbm, out, rowbuf, onesbuf, degsh, sem):
    del sem
    c = lax.axis_index("c")
    s = lax.axis_index("s")
    pltpu.sync_copy(
        zeros8_hbm.at[pl.ds(s * zrows, zrows)],
        degsh.at[pl.ds(s * zrows, zrows)],
    )
    pltpu.sync_copy(ones_hbm, onesbuf)
    plsc.subcore_barrier()

    def body(i, carry):
      r = (c * 16 + s) * ept + i * dch
      pltpu.sync_copy(rows_flat.at[pl.ds(r, dch)], rowbuf)
      pltpu.sync_copy(onesbuf, degsh.at[rowbuf], add=True)
      return carry

    lax.fori_loop(0, nit, body, 0)
    plsc.subcore_barrier()
    pltpu.sync_copy(
        degsh.at[pl.ds(s * zrows, zrows)], out.at[c].at[pl.ds(s * zrows, zrows)]
    )

  return deg_kernel


def _make_agg_kernel(n_acc, ep):
  ept = ep // 16
  chunk = _pick_chunk(ept, 208)
  nit = ept // chunk
  zrows = n_acc // 16
  mesh = plsc.VectorSubcoreMesh(core_axis_name="c", subcore_axis_name="s")

  @functools.partial(
      pl.kernel,
      out_type=jax.ShapeDtypeStruct((n_acc, 2 * _HALF), jnp.float32),
      mesh=mesh,
      compiler_params=pltpu.CompilerParams(use_tc_tiling_on_sc=False),
      scratch_types=[
          pltpu.VMEM((6, chunk), jnp.int32),
          pltpu.VMEM((6, chunk), jnp.int32),
          pltpu.VMEM((4, chunk, _HALF), jnp.float32),
          pltpu.VMEM_SHARED((n_acc, _HALF), jnp.float32),
          pltpu.SemaphoreType.DMA,
          pltpu.SemaphoreType.DMA,
          pltpu.SemaphoreType.DMA,
      ],
  )
  def agg_kernel(ego_flat, cols2f, rows_flat, zeros_hbm, out,
                 colbuf, rowbuf, gbuf, aggsh, semi, semg, sems):
    c = lax.axis_index("c")
    s = lax.axis_index("s")
    base = s * ept
    pltpu.sync_copy(
        zeros_hbm.at[pl.ds(s * zrows, zrows)], aggsh.at[pl.ds(s * zrows, zrows)]
    )
    plsc.subcore_barrier()

    def fire_idx(g):
      i6 = lax.rem(g, 6)
      pltpu.async_copy(
          cols2f.at[c].at[pl.ds(base + g * chunk, chunk)], colbuf.at[i6], semi)
      pltpu.async_copy(
          rows_flat.at[pl.ds(base + g * chunk, chunk)], rowbuf.at[i6], semi)

    def wait_idx():
      pltpu.make_async_copy(
          rows_flat.at[pl.ds(0, chunk)], colbuf.at[0], semi).wait()
      pltpu.make_async_copy(
          rows_flat.at[pl.ds(0, chunk)], rowbuf.at[0], semi).wait()

    def fire_gather(g):
      pltpu.async_copy(
          ego_flat.at[colbuf.at[lax.rem(g, 6)]], gbuf.at[lax.rem(g, 4)], semg)

    def wait_gather():
      pltpu.make_async_copy(
          zeros_hbm.at[pl.ds(0, chunk)], gbuf.at[0], semg).wait()

    def fire_scatter(g):
      pltpu.async_copy(
          gbuf.at[lax.rem(g, 4)], aggsh.at[rowbuf.at[lax.rem(g, 6)]], sems,
          add=True)

    def wait_scatter():
      pltpu.make_async_copy(
          zeros_hbm.at[pl.ds(0, chunk)], gbuf.at[0], sems).wait()

    fire_idx(0)
    fire_idx(1)
    fire_idx(2)
    wait_idx()
    fire_gather(0)
    wait_idx()
    fire_gather(1)

    def body(g, carry):
      @pl.when(g >= 2)
      def _():
        wait_scatter()

      @pl.when(g + 3 < nit)
      def _():
        fire_idx(g + 3)

      @pl.when(g + 2 < nit)
      def _():
        wait_idx()
        fire_gather(g + 2)

      wait_gather()
      fire_scatter(g)
      return carry

    lax.fori_loop(0, nit, body, 0)
    wait_scatter()
    wait_scatter()

    plsc.subcore_barrier()
    pltpu.sync_copy(
        aggsh.at[pl.ds(s * zrows, zrows)],
        out.at[pl.ds(s * zrows, zrows), pl.ds(c * _HALF, _HALF)],
    )

  return agg_kernel


def _layer_math(ego, agg, deg, wg_ref, bg_ref, wb_ref, bb_ref):
  inv = 1.0 / (deg + 1.0)
  side = (agg + ego) * inv
  se = jnp.dot(side, wg_ref[...], preferred_element_type=jnp.float32) + bg_ref[...]
  se = jnp.where(se >= 0.0, se, 0.01 * se)
  be = (
      jnp.dot(ego * side, wb_ref[...], preferred_element_type=jnp.float32)
      + bb_ref[...]
  )
  be = jnp.where(be >= 0.0, be, 0.01 * be)
  e = se + be
  nrm = jnp.sqrt(jnp.sum(e * e, axis=1, keepdims=True))
  nrm = jnp.maximum(nrm, 1e-12)
  return e / nrm


def _tc_update(ego, agg, deg8, wg, bg, wb, bb, block):
  n = ego.shape[0]

  def body(ego_ref, agg_ref, deg_ref, wg_ref, bg_ref, wb_ref, bb_ref, out_ref):
    deg = deg_ref[0, :, 0:1] + deg_ref[1, :, 0:1]
    out_ref[...] = _layer_math(
        ego_ref[...], agg_ref[...], deg, wg_ref, bg_ref, wb_ref, bb_ref)

  return pl.pallas_call(
      body,
      grid=(n // block,),
      in_specs=[
          pl.BlockSpec((block, 64), lambda i: (i, 0)),
          pl.BlockSpec((block, 64), lambda i: (i, 0)),
          pl.BlockSpec((2, block, 8), lambda i: (0, i, 0)),
          pl.BlockSpec((64, 64), lambda i: (0, 0)),
          pl.BlockSpec((1, 64), lambda i: (0, 0)),
          pl.BlockSpec((64, 64), lambda i: (0, 0)),
          pl.BlockSpec((1, 64), lambda i: (0, 0)),
      ],
      out_specs=pl.BlockSpec((block, 64), lambda i: (i, 0)),
      out_shape=jax.ShapeDtypeStruct((n, 64), jnp.float32),
  )(ego, agg, deg8, wg, bg, wb, bb)


def _tc_update_final(ego0, ego1, agg, deg8, wg, bg, wb, bb, block):
  n = ego1.shape[0]

  def body(e0_ref, e1_ref, agg_ref, deg_ref, wg_ref, bg_ref, wb_ref, bb_ref,
           out_ref):
    deg = deg_ref[0, :, 0:1] + deg_ref[1, :, 0:1]
    o = _layer_math(
        e1_ref[...], agg_ref[...], deg, wg_ref, bg_ref, wb_ref, bb_ref)
    out_ref[...] = jnp.concatenate([e0_ref[...], e1_ref[...], o], axis=1)

  return pl.pallas_call(
      body,
      grid=(n // block,),
      in_specs=[
          pl.BlockSpec((block, 64), lambda i: (i, 0)),
          pl.BlockSpec((block, 64), lambda i: (i, 0)),
          pl.BlockSpec((block, 64), lambda i: (i, 0)),
          pl.BlockSpec((2, block, 8), lambda i: (0, i, 0)),
          pl.BlockSpec((64, 64), lambda i: (0, 0)),
          pl.BlockSpec((1, 64), lambda i: (0, 0)),
          pl.BlockSpec((64, 64), lambda i: (0, 0)),
          pl.BlockSpec((1, 64), lambda i: (0, 0)),
      ],
      out_specs=pl.BlockSpec((block, 192), lambda i: (i, 0)),
      out_shape=jax.ShapeDtypeStruct((n, 192), jnp.float32),
  )(ego0, ego1, agg, deg8, wg, bg, wb, bb)


def kernel(edge_index, user_emb, item_emb, W_gc_0, b_gc_0, W_bi_0, b_bi_0,
           W_gc_1, b_gc_1, W_bi_1, b_bi_1):
  n = user_emb.shape[0] + item_emb.shape[0]
  e = edge_index.shape[1]

  ep = e if e % 256 == 0 else -(-e // 16384) * 16384
  pad = ep - e
  if pad:
    rows_flat = jnp.concatenate(
        [edge_index[0], jnp.full((pad,), n, jnp.int32)])
    cols_p = jnp.concatenate([edge_index[1], jnp.zeros((pad,), jnp.int32)])
  else:
    rows_flat = edge_index[0]
    cols_p = edge_index[1]
  cols2f = jnp.stack([2 * cols_p, 2 * cols_p + 1])

  n_acc = -(-(n + 1) // 128) * 128
  zeros_hbm = jnp.zeros((n_acc, _HALF), jnp.float32)
  zeros8_hbm = jnp.zeros((n_acc, 8), jnp.float32)
  ones_hbm = jnp.ones((_pick_chunk(ep // 32, 512), 8), jnp.float32)

  ego0 = jnp.concatenate([user_emb, item_emb], axis=0)

  deg_k = _make_deg_kernel(n_acc, ep)
  agg_k = _make_agg_kernel(n_acc, ep)

  deg8 = deg_k(rows_flat, zeros8_hbm, ones_hbm)

  agg0 = agg_k(ego0.reshape(2 * n, _HALF), cols2f, rows_flat, zeros_hbm)
  ego1 = _tc_update(ego0, agg0, deg8, W_gc_0, b_gc_0, W_bi_0, b_bi_0,
                    block=2000)
  agg1 = agg_k(ego1.reshape(2 * n, _HALF), cols2f, rows_flat, zeros_hbm)
  return _tc_update_final(ego0, ego1, agg1, deg8, W_gc_1, b_gc_1, W_bi_1,
                          b_bi_1, block=2000)


# speedup vs baseline: 30.2959x; 1.0574x over previous
import functools

import jax
import jax.numpy as jnp
from jax import lax
from jax.experimental import pallas as pl
from jax.experimental.pallas import tpu as pltpu
from jax.experimental.pallas import tpu_sc as plsc

_HALF = 32


def _pick_chunk(ept, cap):
  best = 8
  for d in range(8, cap + 1, 8):
    if ept % d == 0:
      best = d
  return best


def _make_deg_kernel(n_acc, ep):
  ept = ep // 32
  dch = _pick_chunk(ept, 512)
  nit = ept // dch
  zrows = n_acc // 16
  mesh = plsc.VectorSubcoreMesh(core_axis_name="c", subcore_axis_name="s")

  @functools.partial(
      pl.kernel,
      out_type=jax.ShapeDtypeStruct((2, n_acc, 8), jnp.float32),
      mesh=mesh,
      compiler_params=pltpu.CompilerParams(use_tc_tiling_on_sc=False),
      scratch_types=[
          pltpu.VMEM((6, dch), jnp.int32),
          pltpu.VMEM((dch, 8), jnp.float32),
          pltpu.VMEM_SHARED((n_acc, 8), jnp.float32),
          pltpu.SemaphoreType.DMA,
          pltpu.SemaphoreType.DMA,
      ],
  )
  def deg_kernel(rows_flat, zeros8_hbm, ones_hbm, out, rowbuf, onesbuf, degsh,
                 semi, sems):
    c = lax.axis_index("c")
    s = lax.axis_index("s")
    base = (c * 16 + s) * ept
    pltpu.sync_copy(
        zeros8_hbm.at[pl.ds(s * zrows, zrows)],
        degsh.at[pl.ds(s * zrows, zrows)],
    )
    pltpu.sync_copy(ones_hbm, onesbuf)
    plsc.subcore_barrier()

    def fire_idx(g):
      pltpu.async_copy(
          rows_flat.at[pl.ds(base + g * dch, dch)], rowbuf.at[lax.rem(g, 6)],
          semi)

    def wait_idx():
      pltpu.make_async_copy(
          rows_flat.at[pl.ds(0, dch)], rowbuf.at[0], semi).wait()

    def wait_scatter():
      pltpu.make_async_copy(
          zeros8_hbm.at[pl.ds(0, dch)], onesbuf, sems).wait()

    fire_idx(0)
    fire_idx(1)
    fire_idx(2)

    def body(g, carry):
      @pl.when(g >= 2)
      def _():
        wait_scatter()

      @pl.when(g + 3 < nit)
      def _():
        fire_idx(g + 3)

      wait_idx()
      pltpu.async_copy(
          onesbuf, degsh.at[rowbuf.at[lax.rem(g, 6)]], sems, add=True)
      return carry

    lax.fori_loop(0, nit, body, 0)
    wait_scatter()
    wait_scatter()
    plsc.subcore_barrier()
    pltpu.sync_copy(
        degsh.at[pl.ds(s * zrows, zrows)], out.at[c].at[pl.ds(s * zrows, zrows)]
    )

  return deg_kernel


def _make_agg_kernel(n_acc, ep):
  ept = ep // 16
  chunk = _pick_chunk(ept, 208)
  nit = ept // chunk
  zrows = n_acc // 16
  mesh = plsc.VectorSubcoreMesh(core_axis_name="c", subcore_axis_name="s")

  @functools.partial(
      pl.kernel,
      out_type=jax.ShapeDtypeStruct((n_acc, 2 * _HALF), jnp.float32),
      mesh=mesh,
      compiler_params=pltpu.CompilerParams(use_tc_tiling_on_sc=False),
      scratch_types=[
          pltpu.VMEM((6, chunk), jnp.int32),
          pltpu.VMEM((6, chunk), jnp.int32),
          pltpu.VMEM((4, chunk, _HALF), jnp.float32),
          pltpu.VMEM_SHARED((n_acc, _HALF), jnp.float32),
          pltpu.SemaphoreType.DMA,
          pltpu.SemaphoreType.DMA,
          pltpu.SemaphoreType.DMA,
      ],
  )
  def agg_kernel(ego_flat, cols2f, rows_flat, zeros_hbm, out,
                 colbuf, rowbuf, gbuf, aggsh, semi, semg, sems):
    c = lax.axis_index("c")
    s = lax.axis_index("s")
    base = s * ept
    pltpu.sync_copy(
        zeros_hbm.at[pl.ds(s * zrows, zrows)], aggsh.at[pl.ds(s * zrows, zrows)]
    )
    plsc.subcore_barrier()

    def fire_idx(g):
      i6 = lax.rem(g, 6)
      pltpu.async_copy(
          cols2f.at[c].at[pl.ds(base + g * chunk, chunk)], colbuf.at[i6], semi)
      pltpu.async_copy(
          rows_flat.at[pl.ds(base + g * chunk, chunk)], rowbuf.at[i6], semi)

    def wait_idx():
      pltpu.make_async_copy(
          rows_flat.at[pl.ds(0, chunk)], colbuf.at[0], semi).wait()
      pltpu.make_async_copy(
          rows_flat.at[pl.ds(0, chunk)], rowbuf.at[0], semi).wait()

    def fire_gather(g):
      pltpu.async_copy(
          ego_flat.at[colbuf.at[lax.rem(g, 6)]], gbuf.at[lax.rem(g, 4)], semg)

    def wait_gather():
      pltpu.make_async_copy(
          zeros_hbm.at[pl.ds(0, chunk)], gbuf.at[0], semg).wait()

    def fire_scatter(g):
      pltpu.async_copy(
          gbuf.at[lax.rem(g, 4)], aggsh.at[rowbuf.at[lax.rem(g, 6)]], sems,
          add=True)

    def wait_scatter():
      pltpu.make_async_copy(
          zeros_hbm.at[pl.ds(0, chunk)], gbuf.at[0], sems).wait()

    fire_idx(0)
    fire_idx(1)
    fire_idx(2)
    wait_idx()
    fire_gather(0)
    wait_idx()
    fire_gather(1)

    def body(g, carry):
      @pl.when(g >= 2)
      def _():
        wait_scatter()

      @pl.when(g + 3 < nit)
      def _():
        fire_idx(g + 3)

      @pl.when(g + 2 < nit)
      def _():
        wait_idx()
        fire_gather(g + 2)

      wait_gather()
      fire_scatter(g)
      return carry

    lax.fori_loop(0, nit, body, 0)
    wait_scatter()
    wait_scatter()

    plsc.subcore_barrier()
    pltpu.sync_copy(
        aggsh.at[pl.ds(s * zrows, zrows)],
        out.at[pl.ds(s * zrows, zrows), pl.ds(c * _HALF, _HALF)],
    )

  return agg_kernel


def _layer_math(ego, agg, deg, wg_ref, bg_ref, wb_ref, bb_ref):
  inv = 1.0 / (deg + 1.0)
  side = (agg + ego) * inv
  se = jnp.dot(side, wg_ref[...], preferred_element_type=jnp.float32) + bg_ref[...]
  se = jnp.where(se >= 0.0, se, 0.01 * se)
  be = (
      jnp.dot(ego * side, wb_ref[...], preferred_element_type=jnp.float32)
      + bb_ref[...]
  )
  be = jnp.where(be >= 0.0, be, 0.01 * be)
  e = se + be
  nrm = jnp.sqrt(jnp.sum(e * e, axis=1, keepdims=True))
  nrm = jnp.maximum(nrm, 1e-12)
  return e / nrm


def _tc_update(ego, agg, deg8, wg, bg, wb, bb, block):
  n = ego.shape[0]

  def body(ego_ref, agg_ref, deg_ref, wg_ref, bg_ref, wb_ref, bb_ref, out_ref):
    deg = deg_ref[0, :, 0:1] + deg_ref[1, :, 0:1]
    out_ref[...] = _layer_math(
        ego_ref[...], agg_ref[...], deg, wg_ref, bg_ref, wb_ref, bb_ref)

  return pl.pallas_call(
      body,
      grid=(n // block,),
      in_specs=[
          pl.BlockSpec((block, 64), lambda i: (i, 0)),
          pl.BlockSpec((block, 64), lambda i: (i, 0)),
          pl.BlockSpec((2, block, 8), lambda i: (0, i, 0)),
          pl.BlockSpec((64, 64), lambda i: (0, 0)),
          pl.BlockSpec((1, 64), lambda i: (0, 0)),
          pl.BlockSpec((64, 64), lambda i: (0, 0)),
          pl.BlockSpec((1, 64), lambda i: (0, 0)),
      ],
      out_specs=pl.BlockSpec((block, 64), lambda i: (i, 0)),
      out_shape=jax.ShapeDtypeStruct((n, 64), jnp.float32),
  )(ego, agg, deg8, wg, bg, wb, bb)


def _tc_update_final(ego0, ego1, agg, deg8, wg, bg, wb, bb, block):
  n = ego1.shape[0]

  def body(e0_ref, e1_ref, agg_ref, deg_ref, wg_ref, bg_ref, wb_ref, bb_ref,
           out_ref):
    deg = deg_ref[0, :, 0:1] + deg_ref[1, :, 0:1]
    o = _layer_math(
        e1_ref[...], agg_ref[...], deg, wg_ref, bg_ref, wb_ref, bb_ref)
    out_ref[...] = jnp.concatenate([e0_ref[...], e1_ref[...], o], axis=1)

  return pl.pallas_call(
      body,
      grid=(n // block,),
      in_specs=[
          pl.BlockSpec((block, 64), lambda i: (i, 0)),
          pl.BlockSpec((block, 64), lambda i: (i, 0)),
          pl.BlockSpec((block, 64), lambda i: (i, 0)),
          pl.BlockSpec((2, block, 8), lambda i: (0, i, 0)),
          pl.BlockSpec((64, 64), lambda i: (0, 0)),
          pl.BlockSpec((1, 64), lambda i: (0, 0)),
          pl.BlockSpec((64, 64), lambda i: (0, 0)),
          pl.BlockSpec((1, 64), lambda i: (0, 0)),
      ],
      out_specs=pl.BlockSpec((block, 192), lambda i: (i, 0)),
      out_shape=jax.ShapeDtypeStruct((n, 192), jnp.float32),
  )(ego0, ego1, agg, deg8, wg, bg, wb, bb)


def kernel(edge_index, user_emb, item_emb, W_gc_0, b_gc_0, W_bi_0, b_bi_0,
           W_gc_1, b_gc_1, W_bi_1, b_bi_1):
  n = user_emb.shape[0] + item_emb.shape[0]
  e = edge_index.shape[1]

  ep = e if e % 256 == 0 else -(-e // 16384) * 16384
  pad = ep - e
  if pad:
    rows_flat = jnp.concatenate(
        [edge_index[0], jnp.full((pad,), n, jnp.int32)])
    cols_p = jnp.concatenate([edge_index[1], jnp.zeros((pad,), jnp.int32)])
  else:
    rows_flat = edge_index[0]
    cols_p = edge_index[1]
  cols2f = jnp.stack([2 * cols_p, 2 * cols_p + 1])

  n_acc = -(-(n + 1) // 128) * 128
  zeros_hbm = jnp.zeros((n_acc, _HALF), jnp.float32)
  zeros8_hbm = jnp.zeros((n_acc, 8), jnp.float32)
  ones_hbm = jnp.ones((_pick_chunk(ep // 32, 512), 8), jnp.float32)

  ego0 = jnp.concatenate([user_emb, item_emb], axis=0)

  deg_k = _make_deg_kernel(n_acc, ep)
  agg_k = _make_agg_kernel(n_acc, ep)

  deg8 = deg_k(rows_flat, zeros8_hbm, ones_hbm)

  agg0 = agg_k(ego0.reshape(2 * n, _HALF), cols2f, rows_flat, zeros_hbm)
  ego1 = _tc_update(ego0, agg0, deg8, W_gc_0, b_gc_0, W_bi_0, b_bi_0,
                    block=2000)
  agg1 = agg_k(ego1.reshape(2 * n, _HALF), cols2f, rows_flat, zeros_hbm)
  return _tc_update_final(ego0, ego1, agg1, deg8, W_gc_1, b_gc_1, W_bi_1,
                          b_bi_1, block=2000)


# speedup vs baseline: 30.3966x; 1.0033x over previous
import functools

import jax
import jax.numpy as jnp
from jax import lax
from jax.experimental import pallas as pl
from jax.experimental.pallas import tpu as pltpu
from jax.experimental.pallas import tpu_sc as plsc

_HALF = 32


def _pick_chunk(ept, cap):
  best = 8
  for d in range(8, cap + 1, 8):
    if ept % d == 0:
      best = d
  return best


def _make_deg_kernel(n_acc, ep):
  ept = ep // 32
  dch = _pick_chunk(ept, 512)
  nit = ept // dch
  zrows = n_acc // 16
  mesh = plsc.VectorSubcoreMesh(core_axis_name="c", subcore_axis_name="s")

  @functools.partial(
      pl.kernel,
      out_type=jax.ShapeDtypeStruct((2, n_acc, 8), jnp.float32),
      mesh=mesh,
      compiler_params=pltpu.CompilerParams(use_tc_tiling_on_sc=False),
      scratch_types=[
          pltpu.VMEM((6, dch), jnp.int32),
          pltpu.VMEM((dch, 8), jnp.float32),
          pltpu.VMEM_SHARED((n_acc, 8), jnp.float32),
          pltpu.SemaphoreType.DMA,
          pltpu.SemaphoreType.DMA,
      ],
  )
  def deg_kernel(rows_flat, zeros8_hbm, ones_hbm, out, rowbuf, onesbuf, degsh,
                 semi, sems):
    c = lax.axis_index("c")
    s = lax.axis_index("s")
    base = (c * 16 + s) * ept
    pltpu.sync_copy(
        zeros8_hbm.at[pl.ds(s * zrows, zrows)],
        degsh.at[pl.ds(s * zrows, zrows)],
    )
    pltpu.sync_copy(ones_hbm, onesbuf)
    plsc.subcore_barrier()

    def fire_idx(g):
      pltpu.async_copy(
          rows_flat.at[pl.ds(base + g * dch, dch)], rowbuf.at[lax.rem(g, 6)],
          semi)

    def wait_idx():
      pltpu.make_async_copy(
          rows_flat.at[pl.ds(0, dch)], rowbuf.at[0], semi).wait()

    def wait_scatter():
      pltpu.make_async_copy(
          zeros8_hbm.at[pl.ds(0, dch)], onesbuf, sems).wait()

    fire_idx(0)
    fire_idx(1)
    fire_idx(2)

    def body(g, carry):
      @pl.when(g >= 2)
      def _():
        wait_scatter()

      @pl.when(g + 3 < nit)
      def _():
        fire_idx(g + 3)

      wait_idx()
      pltpu.async_copy(
          onesbuf, degsh.at[rowbuf.at[lax.rem(g, 6)]], sems, add=True)
      return carry

    lax.fori_loop(0, nit, body, 0)
    wait_scatter()
    wait_scatter()
    plsc.subcore_barrier()
    pltpu.sync_copy(
        degsh.at[pl.ds(s * zrows, zrows)], out.at[c].at[pl.ds(s * zrows, zrows)]
    )

  return deg_kernel


def _make_agg_kernel(n_acc, ep):
  ept = ep // 16
  chunk = _pick_chunk(ept, 208)
  nit = ept // chunk
  zrows = n_acc // 16
  mesh = plsc.VectorSubcoreMesh(core_axis_name="c", subcore_axis_name="s")

  @functools.partial(
      pl.kernel,
      out_type=jax.ShapeDtypeStruct((n_acc, 2 * _HALF), jnp.float32),
      mesh=mesh,
      compiler_params=pltpu.CompilerParams(use_tc_tiling_on_sc=False),
      scratch_types=[
          pltpu.VMEM((6, chunk), jnp.int32),
          pltpu.VMEM((6, chunk), jnp.int32),
          pltpu.VMEM((4, chunk, _HALF), jnp.float32),
          pltpu.VMEM_SHARED((n_acc, _HALF), jnp.float32),
          pltpu.SemaphoreType.DMA,
          pltpu.SemaphoreType.DMA,
          pltpu.SemaphoreType.DMA,
      ],
  )
  def agg_kernel(ego_flat, cols2f, rows_flat, zeros_hbm, out,
                 colbuf, rowbuf, gbuf, aggsh, semi, semg, sems):
    c = lax.axis_index("c")
    s = lax.axis_index("s")
    base = s * ept
    pltpu.sync_copy(
        zeros_hbm.at[pl.ds(s * zrows, zrows)], aggsh.at[pl.ds(s * zrows, zrows)]
    )
    plsc.subcore_barrier()

    def fire_idx(g):
      i6 = lax.rem(g, 6)
      pltpu.async_copy(
          cols2f.at[c].at[pl.ds(base + g * chunk, chunk)], colbuf.at[i6], semi)
      pltpu.async_copy(
          rows_flat.at[pl.ds(base + g * chunk, chunk)], rowbuf.at[i6], semi)

    def wait_idx():
      pltpu.make_async_copy(
          rows_flat.at[pl.ds(0, chunk)], colbuf.at[0], semi).wait()
      pltpu.make_async_copy(
          rows_flat.at[pl.ds(0, chunk)], rowbuf.at[0], semi).wait()

    def fire_gather(g):
      pltpu.async_copy(
          ego_flat.at[colbuf.at[lax.rem(g, 6)]], gbuf.at[lax.rem(g, 4)], semg)

    def wait_gather():
      pltpu.make_async_copy(
          zeros_hbm.at[pl.ds(0, chunk)], gbuf.at[0], semg).wait()

    def fire_scatter(g):
      pltpu.async_copy(
          gbuf.at[lax.rem(g, 4)], aggsh.at[rowbuf.at[lax.rem(g, 6)]], sems,
          add=True)

    def wait_scatter():
      pltpu.make_async_copy(
          zeros_hbm.at[pl.ds(0, chunk)], gbuf.at[0], sems).wait()

    fire_idx(0)
    fire_idx(1)
    fire_idx(2)
    wait_idx()
    fire_gather(0)
    wait_idx()
    fire_gather(1)

    def body(g, carry):
      @pl.when(g >= 2)
      def _():
        wait_scatter()

      @pl.when(g + 3 < nit)
      def _():
        fire_idx(g + 3)

      @pl.when(g + 2 < nit)
      def _():
        wait_idx()
        fire_gather(g + 2)

      wait_gather()
      fire_scatter(g)
      return carry

    lax.fori_loop(0, nit, body, 0)
    wait_scatter()
    wait_scatter()

    plsc.subcore_barrier()
    pltpu.sync_copy(
        aggsh.at[pl.ds(s * zrows, zrows)],
        out.at[pl.ds(s * zrows, zrows), pl.ds(c * _HALF, _HALF)],
    )

  return agg_kernel


def _layer_math(ego, agg, deg, wg_ref, bg_ref, wb_ref, bb_ref):
  inv = 1.0 / (deg + 1.0)
  side = (agg + ego) * inv
  se = jnp.dot(side, wg_ref[...], preferred_element_type=jnp.float32) + bg_ref[...]
  se = jnp.where(se >= 0.0, se, 0.01 * se)
  be = (
      jnp.dot(ego * side, wb_ref[...], preferred_element_type=jnp.float32)
      + bb_ref[...]
  )
  be = jnp.where(be >= 0.0, be, 0.01 * be)
  e = se + be
  nrm = jnp.sqrt(jnp.sum(e * e, axis=1, keepdims=True))
  nrm = jnp.maximum(nrm, 1e-12)
  return e / nrm


def _tc_update(ego, agg, deg8, wg, bg, wb, bb, block):
  n = ego.shape[0]

  def body(ego_ref, agg_ref, deg_ref, wg_ref, bg_ref, wb_ref, bb_ref, out_ref):
    deg = deg_ref[0, :, 0:1] + deg_ref[1, :, 0:1]
    out_ref[...] = _layer_math(
        ego_ref[...], agg_ref[...], deg, wg_ref, bg_ref, wb_ref, bb_ref)

  return pl.pallas_call(
      body,
      grid=(n // block,),
      in_specs=[
          pl.BlockSpec((block, 64), lambda i: (i, 0)),
          pl.BlockSpec((block, 64), lambda i: (i, 0)),
          pl.BlockSpec((2, block, 8), lambda i: (0, i, 0)),
          pl.BlockSpec((64, 64), lambda i: (0, 0)),
          pl.BlockSpec((1, 64), lambda i: (0, 0)),
          pl.BlockSpec((64, 64), lambda i: (0, 0)),
          pl.BlockSpec((1, 64), lambda i: (0, 0)),
      ],
      out_specs=pl.BlockSpec((block, 64), lambda i: (i, 0)),
      out_shape=jax.ShapeDtypeStruct((n, 64), jnp.float32),
  )(ego, agg, deg8, wg, bg, wb, bb)


def _tc_update_final(ego0, ego1, agg, deg8, wg, bg, wb, bb, block):
  n = ego1.shape[0]

  def body(e0_ref, e1_ref, agg_ref, deg_ref, wg_ref, bg_ref, wb_ref, bb_ref,
           out_ref):
    deg = deg_ref[0, :, 0:1] + deg_ref[1, :, 0:1]
    o = _layer_math(
        e1_ref[...], agg_ref[...], deg, wg_ref, bg_ref, wb_ref, bb_ref)
    out_ref[...] = jnp.concatenate([e0_ref[...], e1_ref[...], o], axis=1)

  return pl.pallas_call(
      body,
      grid=(n // block,),
      in_specs=[
          pl.BlockSpec((block, 64), lambda i: (i, 0)),
          pl.BlockSpec((block, 64), lambda i: (i, 0)),
          pl.BlockSpec((block, 64), lambda i: (i, 0)),
          pl.BlockSpec((2, block, 8), lambda i: (0, i, 0)),
          pl.BlockSpec((64, 64), lambda i: (0, 0)),
          pl.BlockSpec((1, 64), lambda i: (0, 0)),
          pl.BlockSpec((64, 64), lambda i: (0, 0)),
          pl.BlockSpec((1, 64), lambda i: (0, 0)),
      ],
      out_specs=pl.BlockSpec((block, 192), lambda i: (i, 0)),
      out_shape=jax.ShapeDtypeStruct((n, 192), jnp.float32),
  )(ego0, ego1, agg, deg8, wg, bg, wb, bb)


def kernel(edge_index, user_emb, item_emb, W_gc_0, b_gc_0, W_bi_0, b_bi_0,
           W_gc_1, b_gc_1, W_bi_1, b_bi_1):
  n = user_emb.shape[0] + item_emb.shape[0]
  e = edge_index.shape[1]

  ep = e if e % 256 == 0 else -(-e // 16384) * 16384
  pad = ep - e
  if pad:
    rows_flat = jnp.concatenate(
        [edge_index[0], jnp.full((pad,), n, jnp.int32)])
    cols_p = jnp.concatenate([edge_index[1], jnp.zeros((pad,), jnp.int32)])
  else:
    rows_flat = edge_index[0]
    cols_p = edge_index[1]
  cols2f = jnp.stack([2 * cols_p, 2 * cols_p + 1])

  n_acc = -(-(n + 1) // 128) * 128
  zeros_hbm = jnp.zeros((n_acc, _HALF), jnp.float32)
  zeros8_hbm = jnp.zeros((n_acc, 8), jnp.float32)
  ones_hbm = jnp.ones((_pick_chunk(ep // 32, 512), 8), jnp.float32)

  ego0 = jnp.concatenate([user_emb, item_emb], axis=0)

  deg_k = _make_deg_kernel(n_acc, ep)
  agg_k = _make_agg_kernel(n_acc, ep)

  deg8 = deg_k(rows_flat, zeros8_hbm, ones_hbm)

  agg0 = agg_k(ego0.reshape(2 * n, _HALF), cols2f, rows_flat, zeros_hbm)
  ego1 = _tc_update(ego0, agg0, deg8, W_gc_0, b_gc_0, W_bi_0, b_bi_0,
                    block=5000)
  agg1 = agg_k(ego1.reshape(2 * n, _HALF), cols2f, rows_flat, zeros_hbm)
  return _tc_update_final(ego0, ego1, agg1, deg8, W_gc_1, b_gc_1, W_bi_1,
                          b_bi_1, block=5000)
